# baseline (speedup 1.0000x reference)
import functools

import jax
import jax.numpy as jnp
from jax import lax
from jax.experimental import pallas as pl
from jax.experimental.pallas import tpu as pltpu
from jax.experimental.pallas import tpu_sc as plsc

NC = 2
NS = 16
CHUNK = 128


def _sc_mesh():
    return plsc.VectorSubcoreMesh(
        core_axis_name="c", subcore_axis_name="s", num_cores=NC, num_subcores=NS
    )


def _fill_vmem(ref, value):
    v = jnp.full((16,), value, jnp.float32)

    @pl.loop(0, ref.shape[0])
    def _(i):
        @pl.loop(0, ref.shape[1], step=16)
        def _(j):
            ref[i, pl.ds(j, 16)] = v


def _sc_aggregate(h, idx4, n_pad):
    n, d = h.shape
    nq = idx4.shape[0]
    ntiles = NC * NS
    pad_per_sub = n_pad // NS
    zrows = 64

    @functools.partial(
        pl.kernel,
        out_type=jax.ShapeDtypeStruct((NC, n_pad, d), jnp.float32),
        mesh=_sc_mesh(),
        scratch_types=[
            [pltpu.VMEM((8, CHUNK), jnp.int32) for _ in range(2)],
            pltpu.VMEM((CHUNK, d), jnp.float32),
            pltpu.VMEM((zrows, d), jnp.float32),
            pltpu.VMEM_SHARED((n_pad, d), jnp.float32),
            pltpu.SemaphoreType.DMA,
            [pltpu.SemaphoreType.DMA for _ in range(2)],
        ],
    )
    def agg(h_hbm, idx_hbm, out_hbm, idx_v, rows_v, zero_v, acc_sh, sem,
            sem_i):
        cid = lax.axis_index("c")
        sid = lax.axis_index("s")
        wid = sid * NC + cid

        _fill_vmem(zero_v, 0.0)
        zbase = sid * pad_per_sub

        @pl.loop(0, pad_per_sub, step=zrows)
        def _(r):
            pltpu.sync_copy(zero_v, acc_sh.at[pl.ds(zbase + r, zrows)])

        plsc.subcore_barrier()

        pltpu.async_copy(idx_hbm.at[wid], idx_v[0], sem_i[0])

        @pl.loop(0, (nq - wid + ntiles - 1) // ntiles, step=2)
        def _(j):
            for u in range(2):
                q = wid + (j + u) * ntiles
                pltpu.make_async_copy(idx_hbm.at[q], idx_v[u],
                                      sem_i[u]).wait()

                @pl.when(q + ntiles < nq)
                def _():
                    pltpu.async_copy(idx_hbm.at[q + ntiles], idx_v[1 - u],
                                     sem_i[1 - u])

                for t in range(4):
                    pltpu.async_copy(h_hbm.at[idx_v[u].at[2 * t]], rows_v,
                                     sem).wait()
                    pltpu.sync_copy(rows_v, acc_sh.at[idx_v[u].at[2 * t + 1]],
                                    add=True)

        plsc.subcore_barrier()

        pltpu.sync_copy(acc_sh.at[pl.ds(zbase, pad_per_sub)],
                        out_hbm.at[cid, pl.ds(zbase, pad_per_sub)])

    return agg(h, idx4)


def _sc_count(dst2, n, n_pad):
    nchunks = dst2.shape[0]
    cpt = nchunks // (NC * NS)
    w = 128
    pad_per_sub = n_pad // NS
    zrows = 64

    @functools.partial(
        pl.kernel,
        out_type=jax.ShapeDtypeStruct((NC, n_pad, w), jnp.float32),
        mesh=_sc_mesh(),
        scratch_types=[
            pltpu.VMEM((cpt, CHUNK), jnp.int32),
            pltpu.VMEM((CHUNK, w), jnp.float32),
            pltpu.VMEM((zrows, w), jnp.float32),
            pltpu.VMEM_SHARED((n_pad, w), jnp.float32),
        ],
    )
    def count(dst_hbm, out_hbm, dst_v, ones_v, zero_v, cnt_sh):
        cid = lax.axis_index("c")
        sid = lax.axis_index("s")
        wid = sid * NC + cid

        row0 = wid * cpt
        pltpu.sync_copy(dst_hbm.at[pl.ds(row0, cpt)], dst_v)

        _fill_vmem(ones_v, 1.0)
        _fill_vmem(zero_v, 0.0)
        zbase = sid * pad_per_sub

        @pl.loop(0, pad_per_sub, step=zrows)
        def _(r):
            pltpu.sync_copy(zero_v, cnt_sh.at[pl.ds(zbase + r, zrows)])

        plsc.subcore_barrier()

        @pl.loop(0, cpt)
        def _(j):
            pltpu.sync_copy(ones_v, cnt_sh.at[dst_v.at[j]], add=True)

        plsc.subcore_barrier()

        pltpu.sync_copy(cnt_sh.at[pl.ds(zbase, pad_per_sub)],
                        out_hbm.at[cid, pl.ds(zbase, pad_per_sub)])

    return count(dst2)


def _tc_combine(h, p0, p1, d0, d1, wr, wn, bi, ai):
    n, d = h.shape
    bm = 1000

    def body(h_ref, p0_ref, p1_ref, d0_ref, d1_ref, wr_ref, wn_ref, b_ref,
             a_ref, o_ref):
        hh = h_ref[...]
        agg = p0_ref[...] + p1_ref[...]
        deg = jnp.maximum(d0_ref[...] + d1_ref[...], 1.0)
        mean = agg / deg
        v = (jnp.dot(hh, wr_ref[...], preferred_element_type=jnp.float32)
             + jnp.dot(mean, wn_ref[...], preferred_element_type=jnp.float32)
             + b_ref[...])
        o_ref[...] = hh + jnp.maximum(v, 0.0) + a_ref[...] * jnp.minimum(v, 0.0)

    return pl.pallas_call(
        body,
        grid=(n // bm,),
        in_specs=[
            pl.BlockSpec((bm, d), lambda i: (i, 0)),
            pl.BlockSpec((bm, d), lambda i: (i, 0)),
            pl.BlockSpec((bm, d), lambda i: (i, 0)),
            pl.BlockSpec((bm, 1), lambda i: (i, 0)),
            pl.BlockSpec((bm, 1), lambda i: (i, 0)),
            pl.BlockSpec((d, d), lambda i: (0, 0)),
            pl.BlockSpec((d, d), lambda i: (0, 0)),
            pl.BlockSpec((1, d), lambda i: (0, 0)),
            pl.BlockSpec((1, d), lambda i: (0, 0)),
        ],
        out_specs=pl.BlockSpec((bm, d), lambda i: (i, 0)),
        out_shape=jax.ShapeDtypeStruct((n, d), jnp.float32),
    )(h, p0, p1, d0, d1, wr, wn, bi, ai)


def kernel(x, edge_index, W_root, W_nei, b, prelu_a):
    n, d = x.shape
    src = edge_index[0]
    dst = edge_index[1]
    e = src.shape[0]
    nconv = W_root.shape[0]

    step = NS * 64
    n_pad = ((n + step - 1) // step) * step
    if n_pad == n:
        n_pad += step

    def junk_rows(m):
        return n + (jnp.arange(m, dtype=jnp.int32) % (n_pad - n))

    gran_c = CHUNK * NC * NS * 8
    e_pad_c = ((e + gran_c - 1) // gran_c) * gran_c
    dst_c = dst if e_pad_c == e else jnp.concatenate([dst, junk_rows(e_pad_c - e)])
    dst2 = dst_c.reshape(e_pad_c // CHUNK, CHUNK)

    gran_a = 4 * CHUNK * NC * NS * 2
    e_pad_a = ((e + gran_a - 1) // gran_a) * gran_a
    if e_pad_a != e:
        src_a = jnp.concatenate([src, jnp.zeros((e_pad_a - e,), jnp.int32)])
        dst_a = jnp.concatenate([dst, junk_rows(e_pad_a - e)])
    else:
        src_a, dst_a = src, dst
    idx4 = jnp.stack(
        [src_a.reshape(-1, 4, CHUNK), dst_a.reshape(-1, 4, CHUNK)], axis=2
    ).reshape(-1, 8, CHUNK)

    cnt = _sc_count(dst2, n, n_pad)
    d0 = cnt[0, :n, :1]
    d1 = cnt[1, :n, :1]

    h = x
    for i in range(nconv):
        p = _sc_aggregate(h, idx4, n_pad)
        if i < nconv - 1:
            ai = jnp.full((1, d), prelu_a[i], jnp.float32)
        else:
            ai = jnp.ones((1, d), jnp.float32)
        h = _tc_combine(h, p[0, :n], p[1, :n], d0, d1, W_root[i], W_nei[i],
                        b[i].reshape(1, d), ai)
    return h


# speedup vs baseline: 2.1376x; 2.1376x over previous
import functools

import jax
import jax.numpy as jnp
from jax import lax
from jax.experimental import pallas as pl
from jax.experimental.pallas import tpu as pltpu
from jax.experimental.pallas import tpu_sc as plsc

NC = 2
NS = 16
CHUNK = 128


def _sc_mesh():
    return plsc.VectorSubcoreMesh(
        core_axis_name="c", subcore_axis_name="s", num_cores=NC, num_subcores=NS
    )


def _fill_vmem(ref, value):
    v = jnp.full((16,), value, jnp.float32)

    @pl.loop(0, ref.shape[0])
    def _(i):
        @pl.loop(0, ref.shape[1], step=16)
        def _(j):
            ref[i, pl.ds(j, 16)] = v


def _sc_aggregate(h, idx4, n_pad):
    n, d = h.shape
    nq = idx4.shape[0]
    ntiles = NC * NS
    pad_per_sub = n_pad // NS
    zrows = 64

    @functools.partial(
        pl.kernel,
        out_type=jax.ShapeDtypeStruct((NC, n_pad, d), jnp.float32),
        mesh=_sc_mesh(),
        scratch_types=[
            pltpu.VMEM((8, CHUNK), jnp.int32),
            pltpu.VMEM((CHUNK, d), jnp.float32),
            pltpu.VMEM((zrows, d), jnp.float32),
            pltpu.VMEM_SHARED((n_pad, d), jnp.float32),
            pltpu.SemaphoreType.DMA,
        ],
    )
    def agg(h_hbm, idx_hbm, out_hbm, idx_v, rows_v, zero_v, acc_sh, sem):
        cid = lax.axis_index("c")
        sid = lax.axis_index("s")
        wid = sid * NC + cid

        _fill_vmem(zero_v, 0.0)
        zbase = sid * pad_per_sub

        @pl.loop(0, pad_per_sub, step=zrows)
        def _(r):
            pltpu.sync_copy(zero_v, acc_sh.at[pl.ds(zbase + r, zrows)])

        plsc.subcore_barrier()

        @pl.loop(wid, nq, step=ntiles)
        def _(q):
            pltpu.sync_copy(idx_hbm.at[q], idx_v)
            for t in range(4):
                pltpu.async_copy(h_hbm.at[idx_v.at[2 * t]], rows_v,
                                 sem).wait()
                pltpu.sync_copy(rows_v, acc_sh.at[idx_v.at[2 * t + 1]],
                                add=True)

        plsc.subcore_barrier()

        pltpu.sync_copy(acc_sh.at[pl.ds(zbase, pad_per_sub)],
                        out_hbm.at[cid, pl.ds(zbase, pad_per_sub)])

    return agg(h, idx4)


def _sc_count(dst2, n, n_pad):
    nchunks = dst2.shape[0]
    cpt = nchunks // (NC * NS)
    w = 128
    pad_per_sub = n_pad // NS
    zrows = 64

    @functools.partial(
        pl.kernel,
        out_type=jax.ShapeDtypeStruct((NC, n_pad, w), jnp.float32),
        mesh=_sc_mesh(),
        scratch_types=[
            pltpu.VMEM((cpt, CHUNK), jnp.int32),
            pltpu.VMEM((CHUNK, w), jnp.float32),
            pltpu.VMEM((zrows, w), jnp.float32),
            pltpu.VMEM_SHARED((n_pad, w), jnp.float32),
        ],
    )
    def count(dst_hbm, out_hbm, dst_v, ones_v, zero_v, cnt_sh):
        cid = lax.axis_index("c")
        sid = lax.axis_index("s")
        wid = sid * NC + cid

        row0 = wid * cpt
        pltpu.sync_copy(dst_hbm.at[pl.ds(row0, cpt)], dst_v)

        _fill_vmem(ones_v, 1.0)
        _fill_vmem(zero_v, 0.0)
        zbase = sid * pad_per_sub

        @pl.loop(0, pad_per_sub, step=zrows)
        def _(r):
            pltpu.sync_copy(zero_v, cnt_sh.at[pl.ds(zbase + r, zrows)])

        plsc.subcore_barrier()

        @pl.loop(0, cpt)
        def _(j):
            pltpu.sync_copy(ones_v, cnt_sh.at[dst_v.at[j]], add=True)

        plsc.subcore_barrier()

        pltpu.sync_copy(cnt_sh.at[pl.ds(zbase, pad_per_sub)],
                        out_hbm.at[cid, pl.ds(zbase, pad_per_sub)])

    return count(dst2)


def _tc_combine(h, p0, p1, d0, d1, wr, wn, bi, ai):
    n, d = h.shape
    bm = 1000

    def body(h_ref, p0_ref, p1_ref, d0_ref, d1_ref, wr_ref, wn_ref, b_ref,
             a_ref, o_ref):
        hh = h_ref[...]
        agg = p0_ref[...] + p1_ref[...]
        deg = jnp.maximum(d0_ref[...] + d1_ref[...], 1.0)
        mean = agg / deg
        v = (jnp.dot(hh, wr_ref[...], preferred_element_type=jnp.float32)
             + jnp.dot(mean, wn_ref[...], preferred_element_type=jnp.float32)
             + b_ref[...])
        o_ref[...] = hh + jnp.maximum(v, 0.0) + a_ref[...] * jnp.minimum(v, 0.0)

    return pl.pallas_call(
        body,
        grid=(n // bm,),
        in_specs=[
            pl.BlockSpec((bm, d), lambda i: (i, 0)),
            pl.BlockSpec((bm, d), lambda i: (i, 0)),
            pl.BlockSpec((bm, d), lambda i: (i, 0)),
            pl.BlockSpec((bm, 1), lambda i: (i, 0)),
            pl.BlockSpec((bm, 1), lambda i: (i, 0)),
            pl.BlockSpec((d, d), lambda i: (0, 0)),
            pl.BlockSpec((d, d), lambda i: (0, 0)),
            pl.BlockSpec((1, d), lambda i: (0, 0)),
            pl.BlockSpec((1, d), lambda i: (0, 0)),
        ],
        out_specs=pl.BlockSpec((bm, d), lambda i: (i, 0)),
        out_shape=jax.ShapeDtypeStruct((n, d), jnp.float32),
    )(h, p0, p1, d0, d1, wr, wn, bi, ai)


def kernel(x, edge_index, W_root, W_nei, b, prelu_a):
    n, d = x.shape
    src = edge_index[0]
    dst = edge_index[1]
    e = src.shape[0]
    nconv = W_root.shape[0]

    step = NS * 64
    n_pad = ((n + step - 1) // step) * step
    if n_pad == n:
        n_pad += step

    def junk_rows(m):
        return n + (jnp.arange(m, dtype=jnp.int32) % (n_pad - n))

    gran_c = CHUNK * NC * NS * 8
    e_pad_c = ((e + gran_c - 1) // gran_c) * gran_c
    dst_c = dst if e_pad_c == e else jnp.concatenate([dst, junk_rows(e_pad_c - e)])
    dst2 = dst_c.reshape(e_pad_c // CHUNK, CHUNK)

    e_pad_a = ((e + 4 * CHUNK - 1) // (4 * CHUNK)) * (4 * CHUNK)
    if e_pad_a != e:
        src_a = jnp.concatenate([src, jnp.zeros((e_pad_a - e,), jnp.int32)])
        dst_a = jnp.concatenate([dst, junk_rows(e_pad_a - e)])
    else:
        src_a, dst_a = src, dst
    idx4 = jnp.stack(
        [src_a.reshape(-1, 4, CHUNK), dst_a.reshape(-1, 4, CHUNK)], axis=2
    ).reshape(-1, 8, CHUNK)

    cnt = _sc_count(dst2, n, n_pad)
    d0 = cnt[0, :n, :1]
    d1 = cnt[1, :n, :1]

    h = x
    for i in range(nconv):
        p = _sc_aggregate(h, idx4, n_pad)
        if i < nconv - 1:
            ai = jnp.full((1, d), prelu_a[i], jnp.float32)
        else:
            ai = jnp.ones((1, d), jnp.float32)
        h = _tc_combine(h, p[0, :n], p[1, :n], d0, d1, W_root[i], W_nei[i],
                        b[i].reshape(1, d), ai)
    return h
